# initial kernel scaffold (unmeasured)
import jax
import jax.numpy as jnp
from jax import lax
from jax.experimental import pallas as pl
from jax.experimental.pallas import tpu as pltpu

N_DEV = 4
B, SQ, SKV, D = 2, 256, 256, 512
HL, DH = 4, 64
HD = HL * DH


def kernel(x, Wq, K_ext, V_ext, Wo):
    my = lax.axis_index("i")
    Wq_loc = lax.dynamic_slice_in_dim(Wq, my * HD, HD, axis=1)
    Wo_loc = lax.dynamic_slice_in_dim(Wo, my * HD, HD, axis=0)
    xf = x.reshape(B * SQ, D)
    K = K_ext.transpose(0, 2, 1, 3)
    V = V_ext.transpose(0, 2, 1, 3)

    def body(x_ref, wq_ref, k_ref, v_ref, wo_ref, out_ref,
             comm_ref, send_sems, recv_sems):
        my_pos = lax.axis_index("i")
        left = lax.rem(my_pos + N_DEV - 1, N_DEV)
        right = lax.rem(my_pos + 1, N_DEV)

        barrier_sem = pltpu.get_barrier_semaphore()
        for nbr in (left, right):
            pl.semaphore_signal(
                barrier_sem, inc=1,
                device_id=(nbr,), device_id_type=pl.DeviceIdType.MESH,
            )
        pl.semaphore_wait(barrier_sem, 2)

        q_all = jnp.dot(x_ref[...], wq_ref[...],
                        preferred_element_type=jnp.float32)

        qb = lax.broadcasted_iota(jnp.int32, (SQ, SKV), 0) // 64
        kb = lax.broadcasted_iota(jnp.int32, (SQ, SKV), 1) // 64
        mask = (qb == kb) | (kb == 0) | (lax.rem(qb + kb, 3) == 0)

        for b in range(B):
            ctx_parts = []
            for h in range(HL):
                qh = q_all[b * SQ:(b + 1) * SQ, h * DH:(h + 1) * DH]
                kh = k_ref[b, h]
                s = lax.dot_general(
                    qh, kh, (((1,), (1,)), ((), ())),
                    preferred_element_type=jnp.float32) * 0.125
                s = jnp.where(mask, s, jnp.float32(-1e9))
                m = jnp.max(s, axis=1, keepdims=True)
                w = jnp.exp(s - m)
                w = w / jnp.sum(w, axis=1, keepdims=True)
                ctx_parts.append(jnp.dot(w, v_ref[b, h],
                                         preferred_element_type=jnp.float32))
            ctx = jnp.concatenate(ctx_parts, axis=1)
            partial = jnp.dot(ctx, wo_ref[...],
                              preferred_element_type=jnp.float32)
            comm_ref[0, b] = partial
            out_ref[b] = partial

        for h in range(N_DEV - 1):
            rdma = pltpu.make_async_remote_copy(
                src_ref=comm_ref.at[h],
                dst_ref=comm_ref.at[h + 1],
                send_sem=send_sems.at[h],
                recv_sem=recv_sems.at[h],
                device_id=(right,),
                device_id_type=pl.DeviceIdType.MESH,
            )
            rdma.start()
            rdma.wait()
            for b in range(B):
                out_ref[b] = out_ref[b] + comm_ref[h + 1, b]

    return pl.pallas_call(
        body,
        out_shape=jax.ShapeDtypeStruct((B, SQ, D), jnp.float32),
        in_specs=[pl.BlockSpec(memory_space=pltpu.VMEM)] * 5,
        out_specs=pl.BlockSpec(memory_space=pltpu.VMEM),
        scratch_shapes=[
            pltpu.VMEM((N_DEV, B, SQ, D), jnp.float32),
            pltpu.SemaphoreType.DMA((N_DEV - 1,)),
            pltpu.SemaphoreType.DMA((N_DEV - 1,)),
        ],
        compiler_params=pltpu.CompilerParams(collective_id=0),
    )(xf, Wq_loc, K, V, wo_ref_arg := Wo_loc)


# baseline (device time: 48179 ns/iter reference)
import jax
import jax.numpy as jnp
from jax import lax
from jax.experimental import pallas as pl
from jax.experimental.pallas import tpu as pltpu

N_DEV = 4
B, SQ, SKV, D = 2, 256, 256, 512
HL, DH = 4, 64
HD = HL * DH


def kernel(x, Wq, K_ext, V_ext, Wo):
    my = lax.axis_index("i")
    Wq_loc = lax.dynamic_slice_in_dim(Wq, my * HD, HD, axis=1)
    Wo_loc = lax.dynamic_slice_in_dim(Wo, my * HD, HD, axis=0)
    xf = x.reshape(B * SQ, D)
    K = K_ext.transpose(0, 2, 1, 3)
    V = V_ext.transpose(0, 2, 1, 3)

    def body(x_ref, wq_ref, k_ref, v_ref, wo_ref, out_ref,
             comm_ref, send_sems, recv_sems):
        my_pos = lax.axis_index("i")
        left = lax.rem(my_pos + N_DEV - 1, N_DEV)
        right = lax.rem(my_pos + 1, N_DEV)

        barrier_sem = pltpu.get_barrier_semaphore()
        for nbr in (left, right):
            pl.semaphore_signal(
                barrier_sem, inc=1,
                device_id=(nbr,), device_id_type=pl.DeviceIdType.MESH,
            )
        pl.semaphore_wait(barrier_sem, 2)

        q_all = jnp.dot(x_ref[...], wq_ref[...],
                        preferred_element_type=jnp.float32)

        qb = lax.broadcasted_iota(jnp.int32, (SQ, SKV), 0) // 64
        kb = lax.broadcasted_iota(jnp.int32, (SQ, SKV), 1) // 64
        mask = (qb == kb) | (kb == 0) | (lax.rem(qb + kb, 3) == 0)

        for b in range(B):
            ctx_parts = []
            for h in range(HL):
                qh = q_all[b * SQ:(b + 1) * SQ, h * DH:(h + 1) * DH]
                kh = k_ref[b, h]
                s = lax.dot_general(
                    qh, kh, (((1,), (1,)), ((), ())),
                    preferred_element_type=jnp.float32) * 0.125
                s = jnp.where(mask, s, jnp.float32(-1e9))
                m = jnp.max(s, axis=1, keepdims=True)
                w = jnp.exp(s - m)
                w = w / jnp.sum(w, axis=1, keepdims=True)
                ctx_parts.append(jnp.dot(w, v_ref[b, h],
                                         preferred_element_type=jnp.float32))
            ctx = jnp.concatenate(ctx_parts, axis=1)
            partial = jnp.dot(ctx, wo_ref[...],
                              preferred_element_type=jnp.float32)
            comm_ref[0, b] = partial
            out_ref[b] = partial

        for h in range(N_DEV - 1):
            rdma = pltpu.make_async_remote_copy(
                src_ref=comm_ref.at[h],
                dst_ref=comm_ref.at[h + 1],
                send_sem=send_sems.at[h],
                recv_sem=recv_sems.at[h],
                device_id=(right,),
                device_id_type=pl.DeviceIdType.MESH,
            )
            rdma.start()
            rdma.wait()
            for b in range(B):
                out_ref[b] = out_ref[b] + comm_ref[h + 1, b]

    return pl.pallas_call(
        body,
        out_shape=jax.ShapeDtypeStruct((B, SQ, D), jnp.float32),
        in_specs=[pl.BlockSpec(memory_space=pltpu.VMEM)] * 5,
        out_specs=pl.BlockSpec(memory_space=pltpu.VMEM),
        scratch_shapes=[
            pltpu.VMEM((N_DEV, B, SQ, D), jnp.float32),
            pltpu.SemaphoreType.DMA((N_DEV - 1,)),
            pltpu.SemaphoreType.DMA((N_DEV - 1,)),
        ],
        compiler_params=pltpu.CompilerParams(collective_id=0),
    )(xf, Wq_loc, K, V, Wo_loc)


# device time: 21968 ns/iter; 2.1931x vs baseline; 2.1931x over previous
import jax
import jax.numpy as jnp
from jax import lax
from jax.experimental import pallas as pl
from jax.experimental.pallas import tpu as pltpu

N_DEV = 4
B, SQ, SKV, D = 2, 256, 256, 512
HL, DH = 4, 64
HD = HL * DH


def kernel(x, Wq, K_ext, V_ext, Wo):
    my = lax.axis_index("i")
    Wq_loc = lax.dynamic_slice_in_dim(Wq, my * HD, HD, axis=1)
    xf = x.reshape(B * SQ, D)
    K = K_ext.transpose(0, 2, 1, 3)
    V = V_ext.transpose(0, 2, 1, 3)

    def body(x_ref, wq_ref, k_ref, v_ref, wo_ref, out_ref,
             ctx_mine, ctx_recv, ssems, rsems):
        my_pos = lax.axis_index("i")

        barrier_sem = pltpu.get_barrier_semaphore()
        for d in (1, 2, 3):
            pl.semaphore_signal(
                barrier_sem, inc=1,
                device_id=(lax.rem(my_pos + d, N_DEV),),
                device_id_type=pl.DeviceIdType.MESH,
            )
        pl.semaphore_wait(barrier_sem, 3)

        q_all = jnp.dot(x_ref[...], wq_ref[...],
                        preferred_element_type=jnp.float32)

        qb = lax.broadcasted_iota(jnp.int32, (SQ, SKV), 0) // 64
        kb = lax.broadcasted_iota(jnp.int32, (SQ, SKV), 1) // 64
        mask = (qb == kb) | (kb == 0) | (lax.rem(qb + kb, 3) == 0)

        def make_desc(k, b, dev_offset):
            return pltpu.make_async_remote_copy(
                src_ref=ctx_mine.at[b],
                dst_ref=ctx_recv.at[k, b],
                send_sem=ssems.at[k, b],
                recv_sem=rsems.at[k, b],
                device_id=(lax.rem(my_pos + dev_offset, N_DEV),),
                device_id_type=pl.DeviceIdType.MESH,
            )

        for b in range(B):
            ctx_parts = []
            for h in range(HL):
                qh = q_all[b * SQ:(b + 1) * SQ, h * DH:(h + 1) * DH]
                kh = k_ref[b, h]
                s = lax.dot_general(
                    qh, kh, (((1,), (1,)), ((), ())),
                    preferred_element_type=jnp.float32) * 0.125
                s = jnp.where(mask, s, jnp.float32(-1e9))
                m = jnp.max(s, axis=1, keepdims=True)
                w = jnp.exp(s - m)
                w = w / jnp.sum(w, axis=1, keepdims=True)
                ctx_parts.append(jnp.dot(w, v_ref[b, h],
                                         preferred_element_type=jnp.float32))
            ctx_mine[b] = jnp.concatenate(ctx_parts, axis=1)
            for d in (1, 2, 3):
                make_desc(3 - d, b, d).start()

        wo_my = wo_ref[pl.ds(my_pos * HD, HD), :]
        for b in range(B):
            out_ref[b] = jnp.dot(ctx_mine[b], wo_my,
                                 preferred_element_type=jnp.float32)

        for k in (0, 2, 1):
            origin = lax.rem(my_pos + k + 1, N_DEV)
            wo_k = wo_ref[pl.ds(origin * HD, HD), :]
            for b in range(B):
                make_desc(k, b, k + 1).wait_recv()
                out_ref[b] = out_ref[b] + jnp.dot(
                    ctx_recv[k, b], wo_k,
                    preferred_element_type=jnp.float32)

        for k in range(N_DEV - 1):
            for b in range(B):
                make_desc(k, b, 3 - k).wait_send()

    return pl.pallas_call(
        body,
        out_shape=jax.ShapeDtypeStruct((B, SQ, D), jnp.float32),
        in_specs=[pl.BlockSpec(memory_space=pltpu.VMEM)] * 5,
        out_specs=pl.BlockSpec(memory_space=pltpu.VMEM),
        scratch_shapes=[
            pltpu.VMEM((B, SQ, HD), jnp.float32),
            pltpu.VMEM((N_DEV - 1, B, SQ, HD), jnp.float32),
            pltpu.SemaphoreType.DMA((N_DEV - 1, B)),
            pltpu.SemaphoreType.DMA((N_DEV - 1, B)),
        ],
        compiler_params=pltpu.CompilerParams(collective_id=0),
    )(xf, Wq_loc, K, V, Wo)


# device time: 16378 ns/iter; 2.9417x vs baseline; 1.3413x over previous
import jax
import jax.numpy as jnp
from jax import lax
from jax.experimental import pallas as pl
from jax.experimental.pallas import tpu as pltpu

N_DEV = 4
B, SQ, SKV, D = 2, 256, 256, 512
HL, DH = 4, 64
HD = HL * DH


def kernel(x, Wq, K_ext, V_ext, Wo):
    my = lax.axis_index("i")
    Wq_loc = lax.dynamic_slice_in_dim(Wq, my * HD, HD, axis=1)
    xf = x.reshape(B * SQ, D)
    K = K_ext.transpose(0, 2, 1, 3)
    V = V_ext.transpose(0, 2, 1, 3)

    def body(x_ref, wq_ref, k_ref, v_ref, wo_ref, out_ref,
             ctx_mine, ctx_recv, ssems, rsems):
        my_pos = lax.axis_index("i")

        barrier_sem = pltpu.get_barrier_semaphore()
        for d in (1, 2, 3):
            pl.semaphore_signal(
                barrier_sem, inc=1,
                device_id=(lax.rem(my_pos + d, N_DEV),),
                device_id_type=pl.DeviceIdType.MESH,
            )
        pl.semaphore_wait(barrier_sem, 3)

        q_all = jnp.dot(x_ref[...], wq_ref[...],
                        preferred_element_type=jnp.float32)

        qb = lax.broadcasted_iota(jnp.int32, (SQ, SKV), 0) // 64
        kb = lax.broadcasted_iota(jnp.int32, (SQ, SKV), 1) // 64
        mask = (qb == kb) | (kb == 0) | (lax.rem(qb + kb, 3) == 0)

        def make_desc(k, b, dev_offset):
            return pltpu.make_async_remote_copy(
                src_ref=ctx_mine.at[b],
                dst_ref=ctx_recv.at[k, b],
                send_sem=ssems.at[k, b],
                recv_sem=rsems.at[k, b],
                device_id=(lax.rem(my_pos + dev_offset, N_DEV),),
                device_id_type=pl.DeviceIdType.MESH,
            )

        for b in range(B):
            ctx_parts = []
            for h in range(HL):
                qh = q_all[b * SQ:(b + 1) * SQ, h * DH:(h + 1) * DH]
                kh = k_ref[b, h]
                s = lax.dot_general(
                    qh, kh, (((1,), (1,)), ((), ())),
                    preferred_element_type=jnp.float32) * 0.125
                s = jnp.where(mask, s, jnp.float32(-1e9))
                m = jnp.max(s, axis=1, keepdims=True)
                w = jnp.exp(s - m)
                w = w / jnp.sum(w, axis=1, keepdims=True)
                ctx_parts.append(jnp.dot(w, v_ref[b, h],
                                         preferred_element_type=jnp.float32))
            ctx_mine[b] = jnp.concatenate(ctx_parts, axis=1).astype(jnp.bfloat16)
            for d in (1, 2, 3):
                make_desc(3 - d, b, d).start()

        wo_my = wo_ref[pl.ds(my_pos * HD, HD), :]
        for b in range(B):
            out_ref[b] = jnp.dot(ctx_mine[b].astype(jnp.float32), wo_my,
                                 preferred_element_type=jnp.float32)

        for k in (0, 2, 1):
            origin = lax.rem(my_pos + k + 1, N_DEV)
            wo_k = wo_ref[pl.ds(origin * HD, HD), :]
            for b in range(B):
                make_desc(k, b, k + 1).wait_recv()
                out_ref[b] = out_ref[b] + jnp.dot(
                    ctx_recv[k, b].astype(jnp.float32), wo_k,
                    preferred_element_type=jnp.float32)

        for k in range(N_DEV - 1):
            for b in range(B):
                make_desc(k, b, 3 - k).wait_send()

    return pl.pallas_call(
        body,
        out_shape=jax.ShapeDtypeStruct((B, SQ, D), jnp.float32),
        in_specs=[pl.BlockSpec(memory_space=pltpu.VMEM)] * 5,
        out_specs=pl.BlockSpec(memory_space=pltpu.VMEM),
        scratch_shapes=[
            pltpu.VMEM((B, SQ, HD), jnp.bfloat16),
            pltpu.VMEM((N_DEV - 1, B, SQ, HD), jnp.bfloat16),
            pltpu.SemaphoreType.DMA((N_DEV - 1, B)),
            pltpu.SemaphoreType.DMA((N_DEV - 1, B)),
        ],
        compiler_params=pltpu.CompilerParams(collective_id=0),
    )(xf, Wq_loc, K, V, Wo)
